# baseline (speedup 1.0000x reference)
import functools

import numpy as np
import jax
import jax.numpy as jnp
from jax import lax
from jax.experimental import pallas as pl
from jax.experimental.pallas import tpu as pltpu
from jax.experimental.pallas import tpu_sc as plsc

ALPHA = 4


def _slow_indices(T: int):
    return [int(v) for v in np.linspace(0, T - 1, T // ALPHA).astype(np.int64)]


def _chunks_for_tile(w, per_w, T):
    lo, hi = w * per_w, (w + 1) * per_w
    out = []
    while lo < hi:
        c, t0 = divmod(lo, T)
        n = min(hi - lo, T - t0)
        out.append((c, t0, n))
        lo += n
    return out


def _make_sc_pack(C, T, S, H, W, idx, dtype):
    info = plsc.get_sparse_core_info()
    NC, NS = info.num_cores, info.num_subcores
    NW = NC * NS
    per_w = (C * T) // NW
    R_slow = C * S

    mesh = plsc.VectorSubcoreMesh(core_axis_name="c", subcore_axis_name="s")

    @functools.partial(
        pl.kernel,
        out_type=[
            jax.ShapeDtypeStruct((C, S, H, W), dtype),
            jax.ShapeDtypeStruct((C, T, H, W), dtype),
        ],
        mesh=mesh,
        scratch_types=[pltpu.SemaphoreType.DMA],
    )
    def sc_pack(src_hbm, slow_hbm, fast_hbm, sem):
        wid = lax.axis_index("s") * NC + lax.axis_index("c")
        for w in range(NW):
            @pl.when(wid == w)
            def _(w=w):
                cps = []
                for c, t0, n in _chunks_for_tile(w, per_w, T):
                    cps.append(
                        pltpu.async_copy(
                            src_hbm.at[c, pl.ds(t0, n)],
                            fast_hbm.at[c, pl.ds(t0, n)],
                            sem,
                        )
                    )
                for r in range(w, R_slow, NW):
                    c, k = divmod(r, S)
                    cps.append(
                        pltpu.async_copy(
                            src_hbm.at[c, pl.ds(idx[k], 1)],
                            slow_hbm.at[c, pl.ds(k, 1)],
                            sem,
                        )
                    )
                for cp in cps:
                    cp.wait()

    return sc_pack


def kernel(frames):
    C, T, H, W = frames.shape
    idx = _slow_indices(T)
    S = len(idx)
    slow, fast = _make_sc_pack(C, T, S, H, W, idx, frames.dtype)(frames)
    return (slow, fast)


# speedup vs baseline: 47.5052x; 47.5052x over previous
import numpy as np
import jax
import jax.numpy as jnp
from jax.experimental import pallas as pl
from jax.experimental.pallas import tpu as pltpu

ALPHA = 4
K = 4


def _slow_indices(T: int):
    return [int(v) for v in np.linspace(0, T - 1, T // ALPHA).astype(np.int64)]


def _gather_body(*refs):
    out = refs[1 + K]
    for i in range(K):
        out[:, i : i + 1] = refs[1 + i][...]


def kernel(frames):
    C, T, H, W = frames.shape
    idx = _slow_indices(T)
    S = len(idx)
    idx_arr = jnp.asarray(idx, dtype=jnp.int32)

    def in_spec(i):
        return pl.BlockSpec((C, 1, H, W), lambda g, r, i=i: (0, r[g * K + i], 0, 0))

    grid_spec = pltpu.PrefetchScalarGridSpec(
        num_scalar_prefetch=1,
        grid=(S // K,),
        in_specs=[in_spec(i) for i in range(K)],
        out_specs=pl.BlockSpec((C, K, H, W), lambda g, r: (0, g, 0, 0)),
    )

    slow = pl.pallas_call(
        _gather_body,
        grid_spec=grid_spec,
        out_shape=jax.ShapeDtypeStruct((C, S, H, W), frames.dtype),
    )(idx_arr, *([frames] * K))

    return (slow, frames)


# speedup vs baseline: 54.2974x; 1.1430x over previous
import numpy as np
import jax
import jax.numpy as jnp
from jax.experimental import pallas as pl
from jax.experimental.pallas import tpu as pltpu

ALPHA = 4
B = 8


def _slow_indices(T: int):
    return [int(v) for v in np.linspace(0, T - 1, T // ALPHA).astype(np.int64)]


def kernel(frames):
    C, T, H, W = frames.shape
    idx = _slow_indices(T)
    S = len(idx)
    G = T // B
    P = S // G
    off = np.asarray([idx[k] - (k // P) * B for k in range(S)], dtype=np.int32)
    assert all(0 <= o < B for o in off)

    def body(off_ref, in_ref, slow_ref, fast_ref):
        g = pl.program_id(0)
        fast_ref[...] = in_ref[...]
        for j in range(P):
            o = off_ref[g * P + j]
            slow_ref[:, j : j + 1] = in_ref[:, pl.ds(o, 1)]

    grid_spec = pltpu.PrefetchScalarGridSpec(
        num_scalar_prefetch=1,
        grid=(G,),
        in_specs=[pl.BlockSpec((C, B, H, W), lambda g, r: (0, g, 0, 0))],
        out_specs=[
            pl.BlockSpec((C, P, H, W), lambda g, r: (0, g, 0, 0)),
            pl.BlockSpec((C, B, H, W), lambda g, r: (0, g, 0, 0)),
        ],
    )

    slow, fast = pl.pallas_call(
        body,
        grid_spec=grid_spec,
        out_shape=[
            jax.ShapeDtypeStruct((C, S, H, W), frames.dtype),
            jax.ShapeDtypeStruct((C, T, H, W), frames.dtype),
        ],
    )(jnp.asarray(off), frames)

    return (slow, fast)
